# Initial kernel scaffold; baseline (speedup 1.0000x reference)
import jax
import jax.numpy as jnp
from jax.experimental import pallas as pl


def kernel(hidden_state, obs1, obs2, W, b):
    raise NotImplementedError("write your pallas kernel here")


# speedup vs baseline: 218.1764x; 218.1764x over previous
import functools

import jax
import jax.numpy as jnp
from jax import lax
from jax.experimental import pallas as pl
from jax.experimental.pallas import tpu as pltpu
from jax.experimental.pallas import tpu_sc as plsc

B = 32
T = 512
NGRID = 16
NCELLS = NGRID * NGRID
OUT_DIM = 128
INV_CELL = 1.0 / 0.6
LANES = 16
TRACK_CHUNK = 128
NCHUNKS = T // TRACK_CHUNK
NC = 2
NS = 16


def _occupancy_body(xs_hbm, ys_hbm, occ_hbm, x_v, y_v, occ_v):
    wid = lax.axis_index("s") * NC + lax.axis_index("c")
    base = wid * T
    pltpu.sync_copy(xs_hbm.at[pl.ds(base, T)], x_v)
    pltpu.sync_copy(ys_hbm.at[pl.ds(base, T)], y_v)

    @pl.loop(0, T // LANES)
    def _scale(k):
        sl = pl.ds(k * LANES, LANES)
        x_v[sl] = x_v[sl] * INV_CELL
        y_v[sl] = y_v[sl] * INV_CELL

    ones = jnp.ones((LANES,), jnp.float32)
    zeros = jnp.zeros((LANES,), jnp.float32)
    poison = jnp.full((LANES,), 1e6, jnp.float32)
    lane0 = lax.iota(jnp.int32, LANES) == 0

    for chunk in range(NCHUNKS):
        @pl.loop(0, TRACK_CHUNK)
        def _zero_row(r):
            for c in range(NCELLS // LANES):
                occ_v[r, pl.ds(c * LANES, LANES)] = zeros

        @pl.loop(0, TRACK_CHUNK)
        def _track(il):
            i = chunk * TRACK_CHUNK + il
            idxv = jnp.full((LANES,), i, jnp.int32)
            ilv = jnp.full((LANES,), il, jnp.int32)
            xi = plsc.load_gather(x_v, [idxv])
            yi = plsc.load_gather(y_v, [idxv])
            plsc.store_scatter(x_v, [idxv], poison, mask=lane0)
            xs_ = xi - 8.0
            ys_ = yi - 8.0
            for jc in range(T // LANES):
                sl = pl.ds(jc * LANES, LANES)
                tdx = x_v[sl] - xs_
                tdy = y_v[sl] - ys_
                oxi = tdx.astype(jnp.int32)
                oyi = tdy.astype(jnp.int32)
                m = (tdx >= 0.0) & (tdy >= 0.0) & ((oxi | oyi) < NGRID)
                cell = oxi * NGRID + oyi
                plsc.store_scatter(occ_v, [ilv, cell], ones, mask=m)
            plsc.store_scatter(x_v, [idxv], xi, mask=lane0)

        pltpu.sync_copy(
            occ_v, occ_hbm.at[pl.ds(base + chunk * TRACK_CHUNK, TRACK_CHUNK)]
        )


_occupancy = functools.partial(
    pl.kernel,
    out_type=jax.ShapeDtypeStruct((B * T, NCELLS), jnp.float32),
    mesh=plsc.VectorSubcoreMesh(
        core_axis_name="c", subcore_axis_name="s", num_cores=NC, num_subcores=NS
    ),
    scratch_types=[
        pltpu.VMEM((T,), jnp.float32),
        pltpu.VMEM((T,), jnp.float32),
        pltpu.VMEM((TRACK_CHUNK, NCELLS), jnp.float32),
    ],
    compiler_params=pltpu.CompilerParams(needs_layout_passes=False),
)(_occupancy_body)


def _linear_body(occ_ref, w_ref, b_ref, o_ref):
    acc = jnp.dot(occ_ref[...], w_ref[...], preferred_element_type=jnp.float32)
    o_ref[...] = jnp.maximum(acc + b_ref[...], 0.0)


def _linear(occ, wt, b2):
    m_blk = 512
    return pl.pallas_call(
        _linear_body,
        grid=(B * T // m_blk,),
        in_specs=[
            pl.BlockSpec((m_blk, NCELLS), lambda m: (m, 0)),
            pl.BlockSpec((NCELLS, OUT_DIM), lambda m: (0, 0)),
            pl.BlockSpec((1, OUT_DIM), lambda m: (0, 0)),
        ],
        out_specs=pl.BlockSpec((m_blk, OUT_DIM), lambda m: (m, 0)),
        out_shape=jax.ShapeDtypeStruct((B * T, OUT_DIM), jnp.float32),
    )(occ, wt, b2)


def kernel(hidden_state, obs1, obs2, W, b):
    nan_mask = jnp.isnan(obs2).any(axis=-1, keepdims=True)
    obs = jnp.where(nan_mask, -500.0, obs2)
    xs = obs[..., 0].reshape(B * T)
    ys = obs[..., 1].reshape(B * T)
    occ = _occupancy(xs, ys)
    return _linear(occ, W.T, b.reshape(1, OUT_DIM))


# speedup vs baseline: 218.2209x; 1.0002x over previous
import functools

import jax
import jax.numpy as jnp
from jax import lax
from jax.experimental import pallas as pl
from jax.experimental.pallas import tpu as pltpu
from jax.experimental.pallas import tpu_sc as plsc

B = 32
T = 512
NGRID = 16
NCELLS = NGRID * NGRID
OUT_DIM = 128
INV_CELL = 1.0 / 0.6
LANES = 16
TRACK_CHUNK = 128
NCHUNKS = T // TRACK_CHUNK
NC = 2
NS = 16


def _occupancy_body(xs_hbm, ys_hbm, occ_hbm, x_v, y_v, occ_v):
    wid = lax.axis_index("s") * NC + lax.axis_index("c")
    base = wid * T
    pltpu.sync_copy(xs_hbm.at[pl.ds(base, T)], x_v)
    pltpu.sync_copy(ys_hbm.at[pl.ds(base, T)], y_v)

    @pl.loop(0, T // LANES)
    def _scale(k):
        sl = pl.ds(k * LANES, LANES)
        x_v[sl] = x_v[sl] * INV_CELL
        y_v[sl] = y_v[sl] * INV_CELL

    ones = jnp.ones((LANES,), jnp.float32)
    zeros = jnp.zeros((LANES,), jnp.float32)
    lanes = lax.iota(jnp.int32, LANES)

    for chunk in range(NCHUNKS):
        @pl.loop(0, TRACK_CHUNK)
        def _zero_row(r):
            for c in range(NCELLS // LANES):
                occ_v[r, pl.ds(c * LANES, LANES)] = zeros

        @pl.loop(0, TRACK_CHUNK)
        def _track(il):
            i = chunk * TRACK_CHUNK + il
            idxv = jnp.full((LANES,), i, jnp.int32)
            ilv = jnp.full((LANES,), il, jnp.int32)
            xi = plsc.load_gather(x_v, [idxv])
            yi = plsc.load_gather(y_v, [idxv])
            xs_ = xi - 8.0
            ys_ = yi - 8.0
            self_lane = lanes == jnp.full((LANES,), lax.rem(i, LANES))
            self_chunk = lax.div(i, LANES)
            for jc in range(T // LANES):
                sl = pl.ds(jc * LANES, LANES)
                tdx = x_v[sl] - xs_
                tdy = y_v[sl] - ys_
                oxi = tdx.astype(jnp.int32)
                oyi = tdy.astype(jnp.int32)
                m = (jnp.minimum(tdx, tdy) >= 0.0) & ((oxi | oyi) < NGRID)
                m = m & ~(jnp.full((LANES,), self_chunk == jc) & self_lane)
                cell = oxi * NGRID + oyi
                plsc.store_scatter(occ_v, [ilv, cell], ones, mask=m)

        pltpu.sync_copy(
            occ_v, occ_hbm.at[pl.ds(base + chunk * TRACK_CHUNK, TRACK_CHUNK)]
        )


_occupancy = functools.partial(
    pl.kernel,
    out_type=jax.ShapeDtypeStruct((B * T, NCELLS), jnp.float32),
    mesh=plsc.VectorSubcoreMesh(
        core_axis_name="c", subcore_axis_name="s", num_cores=NC, num_subcores=NS
    ),
    scratch_types=[
        pltpu.VMEM((T,), jnp.float32),
        pltpu.VMEM((T,), jnp.float32),
        pltpu.VMEM((TRACK_CHUNK, NCELLS), jnp.float32),
    ],
    compiler_params=pltpu.CompilerParams(needs_layout_passes=False),
)(_occupancy_body)


def _linear_body(occ_ref, w_ref, b_ref, o_ref):
    acc = jnp.dot(occ_ref[...], w_ref[...], preferred_element_type=jnp.float32)
    o_ref[...] = jnp.maximum(acc + b_ref[...], 0.0)


def _linear(occ, wt, b2):
    m_blk = 512
    return pl.pallas_call(
        _linear_body,
        grid=(B * T // m_blk,),
        in_specs=[
            pl.BlockSpec((m_blk, NCELLS), lambda m: (m, 0)),
            pl.BlockSpec((NCELLS, OUT_DIM), lambda m: (0, 0)),
            pl.BlockSpec((1, OUT_DIM), lambda m: (0, 0)),
        ],
        out_specs=pl.BlockSpec((m_blk, OUT_DIM), lambda m: (m, 0)),
        out_shape=jax.ShapeDtypeStruct((B * T, OUT_DIM), jnp.float32),
    )(occ, wt, b2)


def kernel(hidden_state, obs1, obs2, W, b):
    nan_mask = jnp.isnan(obs2).any(axis=-1, keepdims=True)
    obs = jnp.where(nan_mask, -500.0, obs2)
    xs = obs[..., 0].reshape(B * T)
    ys = obs[..., 1].reshape(B * T)
    occ = _occupancy(xs, ys)
    return _linear(occ, W.T, b.reshape(1, OUT_DIM))


# speedup vs baseline: 335.5643x; 1.5377x over previous
import functools

import jax
import jax.numpy as jnp
from jax import lax
from jax.experimental import pallas as pl
from jax.experimental.pallas import tpu as pltpu
from jax.experimental.pallas import tpu_sc as plsc

B = 32
T = 512
NGRID = 16
NCELLS = NGRID * NGRID
OUT_DIM = 128
INV_CELL = 1.0 / 0.6
LANES = 16
TRACK_CHUNK = 128
NCHUNKS = T // TRACK_CHUNK
NC = 2
NS = 16


def _occupancy_body(xs_hbm, ys_hbm, occ_hbm, x_v, y_v, occ_v):
    wid = lax.axis_index("s") * NC + lax.axis_index("c")
    base = wid * T
    pltpu.sync_copy(xs_hbm.at[pl.ds(base, T)], x_v)
    pltpu.sync_copy(ys_hbm.at[pl.ds(base, T)], y_v)

    @pl.loop(0, T // LANES)
    def _scale(k):
        sl = pl.ds(k * LANES, LANES)
        x_v[sl] = x_v[sl] * INV_CELL
        y_v[sl] = y_v[sl] * INV_CELL

    ones = jnp.ones((LANES,), jnp.float32)
    zeros = jnp.zeros((LANES,), jnp.float32)
    lanes = lax.iota(jnp.int32, LANES)

    for chunk in range(NCHUNKS):
        @plsc.parallel_loop(0, TRACK_CHUNK, unroll=2)
        def _zero_row(r):
            for c in range(NCELLS // LANES):
                occ_v[r, pl.ds(c * LANES, LANES)] = zeros

        @plsc.parallel_loop(0, TRACK_CHUNK, unroll=2)
        def _track(il):
            i = chunk * TRACK_CHUNK + il
            idxv = jnp.full((LANES,), i, jnp.int32)
            ilv = jnp.full((LANES,), il, jnp.int32)
            xi = plsc.load_gather(x_v, [idxv])
            yi = plsc.load_gather(y_v, [idxv])
            xs_ = xi - 8.0
            ys_ = yi - 8.0
            self_lane = lanes == jnp.full((LANES,), lax.rem(i, LANES))
            self_chunk = lax.div(i, LANES)
            for jc in range(T // LANES):
                sl = pl.ds(jc * LANES, LANES)
                tdx = x_v[sl] - xs_
                tdy = y_v[sl] - ys_
                oxi = tdx.astype(jnp.int32)
                oyi = tdy.astype(jnp.int32)
                m = (jnp.minimum(tdx, tdy) >= 0.0) & ((oxi | oyi) < NGRID)
                m = m & ~(jnp.full((LANES,), self_chunk == jc) & self_lane)
                cell = oxi * NGRID + oyi
                plsc.store_scatter(occ_v, [ilv, cell], ones, mask=m)

        pltpu.sync_copy(
            occ_v, occ_hbm.at[pl.ds(base + chunk * TRACK_CHUNK, TRACK_CHUNK)]
        )


_occupancy = functools.partial(
    pl.kernel,
    out_type=jax.ShapeDtypeStruct((B * T, NCELLS), jnp.float32),
    mesh=plsc.VectorSubcoreMesh(
        core_axis_name="c", subcore_axis_name="s", num_cores=NC, num_subcores=NS
    ),
    scratch_types=[
        pltpu.VMEM((T,), jnp.float32),
        pltpu.VMEM((T,), jnp.float32),
        pltpu.VMEM((TRACK_CHUNK, NCELLS), jnp.float32),
    ],
    compiler_params=pltpu.CompilerParams(needs_layout_passes=False),
)(_occupancy_body)


def _linear_body(occ_ref, w_ref, b_ref, o_ref):
    acc = jnp.dot(occ_ref[...], w_ref[...], preferred_element_type=jnp.float32)
    o_ref[...] = jnp.maximum(acc + b_ref[...], 0.0)


def _linear(occ, wt, b2):
    m_blk = 512
    return pl.pallas_call(
        _linear_body,
        grid=(B * T // m_blk,),
        in_specs=[
            pl.BlockSpec((m_blk, NCELLS), lambda m: (m, 0)),
            pl.BlockSpec((NCELLS, OUT_DIM), lambda m: (0, 0)),
            pl.BlockSpec((1, OUT_DIM), lambda m: (0, 0)),
        ],
        out_specs=pl.BlockSpec((m_blk, OUT_DIM), lambda m: (m, 0)),
        out_shape=jax.ShapeDtypeStruct((B * T, OUT_DIM), jnp.float32),
    )(occ, wt, b2)


def kernel(hidden_state, obs1, obs2, W, b):
    nan_mask = jnp.isnan(obs2).any(axis=-1, keepdims=True)
    obs = jnp.where(nan_mask, -500.0, obs2)
    xs = obs[..., 0].reshape(B * T)
    ys = obs[..., 1].reshape(B * T)
    occ = _occupancy(xs, ys)
    return _linear(occ, W.T, b.reshape(1, OUT_DIM))


# speedup vs baseline: 377.5742x; 1.1252x over previous
import functools

import jax
import jax.numpy as jnp
from jax import lax
from jax.experimental import pallas as pl
from jax.experimental.pallas import tpu as pltpu
from jax.experimental.pallas import tpu_sc as plsc

B = 32
T = 512
NGRID = 16
NCELLS = NGRID * NGRID
OUT_DIM = 128
INV_CELL = 1.0 / 0.6
LANES = 16
TRACK_CHUNK = 128
NCHUNKS = T // TRACK_CHUNK
NC = 2
NS = 16


def _occupancy_body(xs_hbm, ys_hbm, occ_hbm, x_v, y_v, occ_v):
    wid = lax.axis_index("s") * NC + lax.axis_index("c")
    base = wid * T
    pltpu.sync_copy(xs_hbm.at[pl.ds(base, T)], x_v)
    pltpu.sync_copy(ys_hbm.at[pl.ds(base, T)], y_v)

    @pl.loop(0, T // LANES)
    def _scale(k):
        sl = pl.ds(k * LANES, LANES)
        x_v[sl] = x_v[sl] * INV_CELL
        y_v[sl] = y_v[sl] * INV_CELL

    ones = jnp.ones((LANES,), jnp.float32)
    zeros = jnp.zeros((LANES,), jnp.float32)
    lanes = lax.iota(jnp.int32, LANES)

    for chunk in range(NCHUNKS):
        @plsc.parallel_loop(0, TRACK_CHUNK, unroll=4)
        def _zero_row(r):
            for c in range(NCELLS // LANES):
                occ_v[r, pl.ds(c * LANES, LANES)] = zeros

        @plsc.parallel_loop(0, TRACK_CHUNK, unroll=4)
        def _track(il):
            i = chunk * TRACK_CHUNK + il
            idxv = jnp.full((LANES,), i, jnp.int32)
            ilv = jnp.full((LANES,), il, jnp.int32)
            xi = plsc.load_gather(x_v, [idxv])
            yi = plsc.load_gather(y_v, [idxv])
            xs_ = xi - 8.0
            ys_ = yi - 8.0
            self_lane = lanes == jnp.full((LANES,), lax.rem(i, LANES))
            self_chunk = lax.div(i, LANES)
            for jc in range(T // LANES):
                sl = pl.ds(jc * LANES, LANES)
                tdx = x_v[sl] - xs_
                tdy = y_v[sl] - ys_
                oxi = tdx.astype(jnp.int32)
                oyi = tdy.astype(jnp.int32)
                m = (jnp.minimum(tdx, tdy) >= 0.0) & ((oxi | oyi) < NGRID)
                m = m & ~(jnp.full((LANES,), self_chunk == jc) & self_lane)
                cell = oxi * NGRID + oyi
                plsc.store_scatter(occ_v, [ilv, cell], ones, mask=m)

        pltpu.sync_copy(
            occ_v, occ_hbm.at[pl.ds(base + chunk * TRACK_CHUNK, TRACK_CHUNK)]
        )


_occupancy = functools.partial(
    pl.kernel,
    out_type=jax.ShapeDtypeStruct((B * T, NCELLS), jnp.float32),
    mesh=plsc.VectorSubcoreMesh(
        core_axis_name="c", subcore_axis_name="s", num_cores=NC, num_subcores=NS
    ),
    scratch_types=[
        pltpu.VMEM((T,), jnp.float32),
        pltpu.VMEM((T,), jnp.float32),
        pltpu.VMEM((TRACK_CHUNK, NCELLS), jnp.float32),
    ],
    compiler_params=pltpu.CompilerParams(needs_layout_passes=False),
)(_occupancy_body)


def _linear_body(occ_ref, w_ref, b_ref, o_ref):
    acc = jnp.dot(occ_ref[...], w_ref[...], preferred_element_type=jnp.float32)
    o_ref[...] = jnp.maximum(acc + b_ref[...], 0.0)


def _linear(occ, wt, b2):
    m_blk = 512
    return pl.pallas_call(
        _linear_body,
        grid=(B * T // m_blk,),
        in_specs=[
            pl.BlockSpec((m_blk, NCELLS), lambda m: (m, 0)),
            pl.BlockSpec((NCELLS, OUT_DIM), lambda m: (0, 0)),
            pl.BlockSpec((1, OUT_DIM), lambda m: (0, 0)),
        ],
        out_specs=pl.BlockSpec((m_blk, OUT_DIM), lambda m: (m, 0)),
        out_shape=jax.ShapeDtypeStruct((B * T, OUT_DIM), jnp.float32),
    )(occ, wt, b2)


def kernel(hidden_state, obs1, obs2, W, b):
    nan_mask = jnp.isnan(obs2).any(axis=-1, keepdims=True)
    obs = jnp.where(nan_mask, -500.0, obs2)
    xs = obs[..., 0].reshape(B * T)
    ys = obs[..., 1].reshape(B * T)
    occ = _occupancy(xs, ys)
    return _linear(occ, W.T, b.reshape(1, OUT_DIM))
